# Initial kernel scaffold; baseline (speedup 1.0000x reference)
import jax
import jax.numpy as jnp
from jax.experimental import pallas as pl


def kernel(input_ids, wte, wpe):
    raise NotImplementedError("write your pallas kernel here")


# speedup vs baseline: 1.0194x; 1.0194x over previous
import functools

import jax
import jax.numpy as jnp
from jax import lax
from jax.experimental import pallas as pl
from jax.experimental.pallas import tpu as pltpu
from jax.experimental.pallas import tpu_sc as plsc

_NUM_CORES = 2
_NUM_SUBCORES = 16
_NUM_WORKERS = _NUM_CORES * _NUM_SUBCORES
_LANES = 16


@functools.partial(jax.jit, static_argnames=("seq_len", "rows_per_chunk"))
def _embed_sc(ids_flat, wte, wpe, *, seq_len, rows_per_chunk):
    n_tok = ids_flat.shape[0]
    n_embd = wte.shape[1]
    rows_per_worker = n_tok // _NUM_WORKERS
    k = rows_per_chunk
    n_chunks = rows_per_worker // k
    lanes_per_row = n_embd // _LANES

    mesh = plsc.VectorSubcoreMesh(
        core_axis_name="c",
        subcore_axis_name="s",
        num_cores=_NUM_CORES,
        num_subcores=_NUM_SUBCORES,
    )

    @functools.partial(
        pl.kernel,
        out_type=jax.ShapeDtypeStruct((n_tok, n_embd), jnp.float32),
        mesh=mesh,
        scratch_types=[
            pltpu.VMEM((k,), jnp.int32),
            pltpu.VMEM((k, n_embd), jnp.float32),
            pltpu.VMEM((k, n_embd), jnp.float32),
            pltpu.SemaphoreType.DMA,
            pltpu.SemaphoreType.DMA,
        ],
    )
    def body(ids_hbm, wte_hbm, wpe_hbm, out_hbm, idx_v, rows_v, wpe_v, sem_g, sem_p):
        wid = lax.axis_index("s") * _NUM_CORES + lax.axis_index("c")

        for c in range(n_chunks):
            base = wid * rows_per_worker + c * k
            s_start = lax.rem(base, seq_len)
            pltpu.sync_copy(ids_hbm.at[pl.ds(base, k)], idx_v)
            gather = pltpu.async_copy(wte_hbm.at[idx_v], rows_v, sem_g)
            pos = pltpu.async_copy(wpe_hbm.at[pl.ds(s_start, k), :], wpe_v, sem_p)
            gather.wait()
            pos.wait()

            def add_row(r):
                for j in range(lanes_per_row):
                    sl = pl.ds(j * _LANES, _LANES)
                    rows_v[r, sl] += wpe_v[r, sl]

            pl.loop(0, k)(add_row)
            pltpu.sync_copy(rows_v, out_hbm.at[pl.ds(base, k), :])

    return body(ids_flat, wte, wpe)


def kernel(input_ids, wte, wpe):
    batch, seq_len = input_ids.shape
    out = _embed_sc(
        input_ids.reshape(-1), wte, wpe, seq_len=seq_len, rows_per_chunk=64
    )
    return out.reshape(batch, seq_len, wte.shape[1])
